# Initial kernel scaffold; baseline (speedup 1.0000x reference)
import jax
import jax.numpy as jnp
from jax.experimental import pallas as pl


def kernel(x_alpha, x_barra, money_flow_graph, sector_graph, params):
    raise NotImplementedError("write your pallas kernel here")


# speedup vs baseline: 5.0020x; 5.0020x over previous
import jax
import jax.numpy as jnp
from jax.experimental import pallas as pl
from jax.experimental.pallas import tpu as pltpu

B, T, N, FA, FB, D, H, O, K = 2, 8, 300, 158, 10, 128, 4, 32, 10
ATH = 0.3
NP = 304
R = 16
NT = NP // R


def _mm(a, b):
    return jnp.dot(a, b, preferred_element_type=jnp.float32)


def _ln(x, g, bb):
    m = jnp.mean(x, axis=-1, keepdims=True)
    v = jnp.mean((x - m) ** 2, axis=-1, keepdims=True)
    return (x - m) / jnp.sqrt(v + 1e-5) * g + bb


def _sigmoid(x):
    return jax.nn.sigmoid(x)


def _prep_body(xb_ref, xa_ref,
               riskW, riskb, riskg, riskbb,
               w1a, w1b, b1,
               mpW, mpb, mnW, mnb,
               inW, inb, ing, inbb,
               h_o, u_o, v_o, msgp_o, msgn_o, he_o):
    xb = xb_ref[0]
    h = jnp.maximum(_ln(_mm(xb, riskW[...]) + riskb[...], riskg[...], riskbb[...]), 0.0)
    h_o[0] = h
    u_o[0] = _mm(h, w1a[...]) + b1[...]
    v_o[0] = _mm(h, w1b[...])
    msgp_o[0] = _mm(h, mpW[...]) + mpb[...]
    msgn_o[0] = _mm(h, mnW[...]) + mnb[...]
    xa = xa_ref[0]
    he_o[0] = jnp.maximum(_ln(_mm(xa, inW[...]) + inb[...], ing[...], inbb[...]), 0.0)


def _att_body(u_ref, v_ref, w2_ref, b2_ref, mfg_ref, msgp_ref, msgn_ref,
              mp_ref, mn_ref):
    t = pl.program_id(1)
    u = u_ref[0]
    v = v_ref[0]
    e = jnp.maximum(u[:, None, :] + v[None, :, :], 0.0)
    s = _mm(e.reshape(R * NP, D), w2_ref[...]).reshape(R, NP)
    att = _sigmoid(s + b2_ref[0, 0])

    rows = t * R + jax.lax.broadcasted_iota(jnp.int32, (R, NP), 0)
    cols = jax.lax.broadcasted_iota(jnp.int32, (R, NP), 1)
    att_nd = jnp.where((rows == cols) | (cols >= N), 0.0, att)

    cur = att_nd
    for _ in range(K):
        m = jnp.max(cur, axis=1, keepdims=True)
        sel = jnp.min(jnp.where(cur == m, cols, NP), axis=1, keepdims=True)
        cur = jnp.where(cols == sel, -1e30, cur)
    maskk = (cur < -1e29).astype(jnp.float32)
    attf = att * maskk

    mfg = mfg_ref[0]
    adjp = attf * (mfg > ATH).astype(jnp.float32)
    adjn = attf * (mfg < -ATH).astype(jnp.float32)
    mp_ref[0] = _mm(adjp, msgp_ref[0])
    mn_ref[0] = _mm(adjn, msgn_ref[0])


def _gru(x, h, Wx, bih, Wh, bhh):
    gi = _mm(x, Wx) + bih
    gh = _mm(h, Wh) + bhh
    r = _sigmoid(gi[:, :D] + gh[:, :D])
    z = _sigmoid(gi[:, D:2 * D] + gh[:, D:2 * D])
    n = jnp.tanh(gi[:, 2 * D:] + r * gh[:, 2 * D:])
    return (1.0 - z) * n + z * h


def _gat_heads(x, adt, asrT, mask):
    outs = []
    for hh in range(H):
        e = adt[:, hh:hh + 1] + asrT[hh:hh + 1, :]
        e = jnp.where(e >= 0.0, e, 0.2 * e)
        e = jnp.where(mask, e, -1e9)
        e = e - jnp.max(e, axis=1, keepdims=True)
        p = jnp.exp(e)
        a = p / jnp.sum(p, axis=1, keepdims=True)
        outs.append(_mm(a, x[:, hh * O:(hh + 1) * O]))
    return outs


def _tail_body(h_ref, mp_ref, mn_ref, he_ref, secT_ref,
               gpWx, gpbih, gpWh, gpbhh,
               gnWx, gnbih, gnWh, gnbhh,
               combWa, combWb, combb,
               g0W, Ws0, Wd0, g0b,
               g1W, Ws1, Wd1, g1b,
               opW, opb,
               gW1a, gW1b, gb1, gW2, gb2,
               fuseW, fuseb, fuseg, fusebb,
               out_ref):
    h = h_ref[0]
    hp = _gru(mp_ref[0], h, gpWx[...], gpbih[...], gpWh[...], gpbhh[...])
    hn = _gru(mn_ref[0], h, gnWx[...], gnbih[...], gnWh[...], gnbhh[...])
    h_imp = _mm(hp, combWa[...]) + _mm(hn, combWb[...]) + combb[...]

    he = he_ref[0]
    rows = jax.lax.broadcasted_iota(jnp.int32, (NP, NP), 0)
    cols = jax.lax.broadcasted_iota(jnp.int32, (NP, NP), 1)
    mask = (secT_ref[0] != 0.0) | (rows == cols)

    x0 = _mm(he, g0W[...])
    adt0 = _mm(x0, Wd0[...])
    asrT0 = jnp.transpose(_mm(x0, Ws0[...]))
    o0 = _gat_heads(x0, adt0, asrT0, mask)
    h1 = jnp.concatenate(o0, axis=1) + g0b[...]
    he2 = he + jnp.where(h1 > 0.0, h1, jnp.exp(jnp.minimum(h1, 0.0)) - 1.0)

    x1 = _mm(he2, g1W[...])
    adt1 = _mm(x1, Wd1[...])
    asrT1 = jnp.transpose(_mm(x1, Ws1[...]))
    o1 = _gat_heads(x1, adt1, asrT1, mask)
    h2 = (o1[0] + o1[1] + o1[2] + o1[3]) * 0.25 + g1b[...]
    h_exp = _mm(h2, opW[...]) + opb[...]

    gmid = jnp.maximum(_mm(h_imp, gW1a[...]) + _mm(h_exp, gW1b[...]) + gb1[...], 0.0)
    gate = _sigmoid(_mm(gmid, gW2[...]) + gb2[...])
    hf = gate * h_imp + (1.0 - gate) * h_exp
    out_ref[0] = jnp.maximum(_ln(_mm(hf, fuseW[...]) + fuseb[...], fuseg[...], fusebb[...]), 0.0)


def _row(a):
    return a.reshape(1, -1)


def _head_blockdiag(a):
    d = jnp.arange(H * O)[:, None] // O == jnp.arange(H)[None, :]
    return a.reshape(H * O, 1) * d.astype(a.dtype)


def kernel(x_alpha, x_barra, money_flow_graph, sector_graph, params):
    p = params
    padr = lambda a: jnp.pad(a, ((0, 0), (0, NP - N), (0, 0)))
    xb = padr(x_barra[:, -1])
    xa = padr(x_alpha[:, -1])
    mfg = jnp.pad(money_flow_graph, ((0, 0), (0, NP - N), (0, NP - N)))
    secT = jnp.pad(jnp.swapaxes(sector_graph, 1, 2), ((0, 0), (0, NP - N), (0, NP - N)))

    bnd = lambda shp: jax.ShapeDtypeStruct(shp, jnp.float32)
    wspec = lambda a: pl.BlockSpec(a.shape, lambda b, *_: (0,) * a.ndim)
    bspec = lambda a: pl.BlockSpec((1,) + a.shape[1:], lambda b, *_: (b,) + (0,) * (a.ndim - 1))

    pw = [p['risk_W'], _row(p['risk_b']), _row(p['risk_g']), _row(p['risk_bb']),
          p['attn_W1'][:D], p['attn_W1'][D:], _row(p['attn_b1']),
          p['msgp_W'], _row(p['msgp_b']), p['msgn_W'], _row(p['msgn_b']),
          p['in_W'], _row(p['in_b']), _row(p['in_g']), _row(p['in_bb'])]
    h, u, v, msgp, msgn, he = pl.pallas_call(
        _prep_body,
        grid=(B,),
        in_specs=[bspec(xb), bspec(xa)] + [wspec(w) for w in pw],
        out_specs=[bspec(jnp.zeros((B, NP, D)))] * 6,
        out_shape=[bnd((B, NP, D))] * 6,
    )(xb, xa, *pw)

    w2row = p['attn_W2']
    b2 = p['attn_b2'].reshape(1, 1)
    tspec = pl.BlockSpec((1, R, NP), lambda b, t: (b, t, 0))
    tdspec = pl.BlockSpec((1, R, D), lambda b, t: (b, t, 0))
    fspec = pl.BlockSpec((1, NP, D), lambda b, t: (b, 0, 0))
    mp, mn = pl.pallas_call(
        _att_body,
        grid=(B, NT),
        in_specs=[tdspec, fspec,
                  pl.BlockSpec(w2row.shape, lambda b, t: (0, 0)),
                  pl.BlockSpec(b2.shape, lambda b, t: (0, 0)),
                  tspec, fspec, fspec],
        out_specs=[tdspec, tdspec],
        out_shape=[bnd((B, NP, D))] * 2,
        compiler_params=pltpu.CompilerParams(
            dimension_semantics=("parallel", "arbitrary")),
    )(u, v, w2row, b2, mfg, msgp, msgn)

    gw = [p['grup_Wx'], _row(p['grup_bih']), p['grup_Wh'], _row(p['grup_bhh']),
          p['grun_Wx'], _row(p['grun_bih']), p['grun_Wh'], _row(p['grun_bhh']),
          p['comb_W'][:D], p['comb_W'][D:], _row(p['comb_b']),
          p['g0_W'], _head_blockdiag(p['g0_as']), _head_blockdiag(p['g0_ad']), _row(p['g0_b']),
          p['g1_W'], _head_blockdiag(p['g1_as']), _head_blockdiag(p['g1_ad']), _row(p['g1_b']),
          p['op_W'], _row(p['op_b']),
          p['gate_W1'][:D], p['gate_W1'][D:], _row(p['gate_b1']),
          p['gate_W2'], _row(p['gate_b2']),
          p['fuse_W'], _row(p['fuse_b']), _row(p['fuse_g']), _row(p['fuse_bb'])]
    out = pl.pallas_call(
        _tail_body,
        grid=(B,),
        in_specs=[bspec(h), bspec(mp), bspec(mn), bspec(he), bspec(secT)]
                 + [wspec(w) for w in gw],
        out_specs=bspec(jnp.zeros((B, NP, D))),
        out_shape=bnd((B, NP, D)),
    )(h, mp, mn, he, secT, *gw)
    return out[:, :N, :]


# speedup vs baseline: 5.9351x; 1.1866x over previous
import functools

import jax
import jax.numpy as jnp
import numpy as np
from jax import lax
from jax.experimental import pallas as pl
from jax.experimental.pallas import tpu as pltpu
from jax.experimental.pallas import tpu_sc as plsc

B, T, N, FA, FB, D, H, O, K = 2, 8, 300, 158, 10, 128, 4, 32, 10
ATH = 0.3
NP = 304
R = 16
NT = NP // R
NROWS = B * NP
NWORK = 32
NRPAD = 768
RPT = NRPAD // NWORK
NCH = NP // 16
BIGI = 1 << 30


def _mm(a, b):
    return jnp.dot(a, b, preferred_element_type=jnp.float32)


def _ln(x, g, bb):
    m = jnp.mean(x, axis=-1, keepdims=True)
    v = jnp.mean((x - m) ** 2, axis=-1, keepdims=True)
    return (x - m) / jnp.sqrt(v + 1e-5) * g + bb


def _sigmoid(x):
    return jax.nn.sigmoid(x)


def _prep_body(xb_ref, xa_ref,
               riskW, riskb, riskg, riskbb,
               w1a, w1b, b1,
               mpW, mpb, mnW, mnb,
               inW, inb, ing, inbb,
               h_o, u_o, v_o, msgp_o, msgn_o, he_o):
    xb = xb_ref[0]
    h = jnp.maximum(_ln(_mm(xb, riskW[...]) + riskb[...], riskg[...], riskbb[...]), 0.0)
    h_o[0] = h
    u_o[0] = _mm(h, w1a[...]) + b1[...]
    v_o[0] = _mm(h, w1b[...])
    msgp_o[0] = _mm(h, mpW[...]) + mpb[...]
    msgn_o[0] = _mm(h, mnW[...]) + mnb[...]
    xa = xa_ref[0]
    he_o[0] = jnp.maximum(_ln(_mm(xa, inW[...]) + inb[...], ing[...], inbb[...]), 0.0)


def _att_body(u_ref, v_ref, w2_ref, b2_ref, att_o):
    u = u_ref[0]
    v = v_ref[0]
    e = jnp.maximum(u[:, None, :] + v[None, :, :], 0.0)
    s = _mm(e.reshape(R * NP, D), w2_ref[...]).reshape(R, NP)
    att_o[0] = _sigmoid(s + b2_ref[0, 0])


def _perm(x, pm):
    return x.at[pm].get(mode="promise_in_bounds")


def _sc_body(att_hbm, mfg_hbm, msgp_hbm, msgn_hbm, mp_hbm, mn_hbm,
             att_v, mfg_v, cur_v, idx_v, rp_v, rn_v, mp_v, mn_v, sem):
    cidx = lax.axis_index("c")
    sidx = lax.axis_index("s")
    wid = sidx * 2 + cidx
    base = wid * RPT
    pltpu.sync_copy(att_hbm.at[pl.ds(base, RPT)], att_v)
    pltpu.sync_copy(mfg_hbm.at[pl.ds(base, RPT)], mfg_v)
    lanes = lax.iota(jnp.int32, 16)

    def _lexmax(mval, midx):
        for sh in (1, 2, 4, 8):
            pm = jnp.bitwise_xor(lanes, sh)
            av = _perm(mval, pm)
            ai = _perm(midx, pm)
            gt = (av > mval) | ((av == mval) & (ai < midx))
            mval = jnp.where(gt, av, mval)
            midx = jnp.where(gt, ai, midx)
        return mval, midx

    def row_body(r, carry):
        g = base + r
        bb = jnp.minimum(g // NP, B - 1)
        di = g - (g // NP) * NP
        rowbase = bb * NP
        for c in range(NCH):
            idxc = c * 16 + lanes
            a = att_v[r, pl.ds(c * 16, 16)]
            cur_v[pl.ds(c * 16, 16)] = jnp.where((idxc == di) | (idxc >= N), 0.0, a)

        jlocv = jnp.zeros((16,), jnp.int32)
        mvalsel = jnp.zeros((16,), jnp.float32)
        prev = jnp.full((16,), -1, jnp.int32)
        for k in range(K):
            def chunk_body(c, mcar):
                mval, midx = mcar
                a = cur_v[pl.ds(c * 16, 16)]
                idxc = c * 16 + lanes
                if k > 0:
                    a = jnp.where(idxc == prev, -1.0, a)
                    cur_v[pl.ds(c * 16, 16)] = a
                gt = (a > mval) | ((a == mval) & (idxc < midx))
                return (jnp.where(gt, a, mval), jnp.where(gt, idxc, midx))
            mval, midx = lax.fori_loop(
                0, NCH, chunk_body,
                (jnp.full((16,), -2.0, jnp.float32), jnp.full((16,), BIGI, jnp.int32)))
            mmaxv, jminv = _lexmax(mval, midx)
            mvalsel = jnp.where(lanes == k, mmaxv, mvalsel)
            jlocv = jnp.where(lanes == k, jminv, jlocv)
            prev = jminv

        mfgvals = plsc.load_gather(mfg_v, [jnp.full((16,), r, jnp.int32), jlocv])
        kmask = lanes < K
        wpv = jnp.where((mfgvals > ATH) & kmask, mvalsel, 0.0)
        wnv = jnp.where((mfgvals < -ATH) & kmask, mvalsel, 0.0)
        idx_v[...] = rowbase + jlocv
        cp = pltpu.async_copy(msgp_hbm.at[idx_v], rp_v, sem)
        cn = pltpu.async_copy(msgn_hbm.at[idx_v], rn_v, sem)
        cp.wait()
        cn.wait()
        for c in range(8):
            accp = jnp.zeros((16,), jnp.float32)
            accn = jnp.zeros((16,), jnp.float32)
            for k in range(K):
                kk = jnp.full((16,), k, jnp.int32)
                accp = accp + _perm(wpv, kk) * rp_v[k, pl.ds(c * 16, 16)]
                accn = accn + _perm(wnv, kk) * rn_v[k, pl.ds(c * 16, 16)]
            mp_v[r, pl.ds(c * 16, 16)] = accp
            mn_v[r, pl.ds(c * 16, 16)] = accn
        return carry

    lax.fori_loop(0, RPT, row_body, 0)
    pltpu.sync_copy(mp_v, mp_hbm.at[pl.ds(base, RPT)])
    pltpu.sync_copy(mn_v, mn_hbm.at[pl.ds(base, RPT)])


def _sc_stage(att2, mfg2, msgp2, msgn2):
    mesh = plsc.VectorSubcoreMesh(core_axis_name="c", subcore_axis_name="s",
                                  num_cores=2, num_subcores=16)
    f = pl.kernel(
        _sc_body,
        out_type=[jax.ShapeDtypeStruct((NRPAD, D), jnp.float32)] * 2,
        mesh=mesh,
        compiler_params=pltpu.CompilerParams(needs_layout_passes=False),
        scratch_types=[
            pltpu.VMEM((RPT, NP), jnp.float32),
            pltpu.VMEM((RPT, NP), jnp.float32),
            pltpu.VMEM((NP,), jnp.float32),
            pltpu.VMEM((16,), jnp.int32),
            pltpu.VMEM((16, D), jnp.float32),
            pltpu.VMEM((16, D), jnp.float32),
            pltpu.VMEM((RPT, D), jnp.float32),
            pltpu.VMEM((RPT, D), jnp.float32),
            pltpu.SemaphoreType.DMA,
        ],
    )
    return f(att2, mfg2, msgp2, msgn2)


def _gat_heads(x, adt, asrT, mask):
    outs = []
    for hh in range(H):
        e = adt[:, hh:hh + 1] + asrT[hh:hh + 1, :]
        e = jnp.where(e >= 0.0, e, 0.2 * e)
        e = jnp.where(mask, e, -1e9)
        e = e - jnp.max(e, axis=1, keepdims=True)
        p = jnp.exp(e)
        a = p / jnp.sum(p, axis=1, keepdims=True)
        outs.append(_mm(a, x[:, hh * O:(hh + 1) * O]))
    return outs


def _gat_body(he_ref, secT_ref,
              g0W, Ws0, Wd0, g0b,
              g1W, Ws1, Wd1, g1b,
              opW, opb,
              hexp_o):
    he = he_ref[0]
    rows = lax.broadcasted_iota(jnp.int32, (NP, NP), 0)
    cols = lax.broadcasted_iota(jnp.int32, (NP, NP), 1)
    mask = (secT_ref[0] != 0.0) | (rows == cols)

    x0 = _mm(he, g0W[...])
    adt0 = _mm(x0, Wd0[...])
    asrT0 = jnp.transpose(_mm(x0, Ws0[...]))
    o0 = _gat_heads(x0, adt0, asrT0, mask)
    h1 = jnp.concatenate(o0, axis=1) + g0b[...]
    he2 = he + jnp.where(h1 > 0.0, h1, jnp.exp(jnp.minimum(h1, 0.0)) - 1.0)

    x1 = _mm(he2, g1W[...])
    adt1 = _mm(x1, Wd1[...])
    asrT1 = jnp.transpose(_mm(x1, Ws1[...]))
    o1 = _gat_heads(x1, adt1, asrT1, mask)
    h2 = (o1[0] + o1[1] + o1[2] + o1[3]) * 0.25 + g1b[...]
    hexp_o[0] = _mm(h2, opW[...]) + opb[...]


def _gru(x, h, Wx, bih, Wh, bhh):
    gi = _mm(x, Wx) + bih
    gh = _mm(h, Wh) + bhh
    r = _sigmoid(gi[:, :D] + gh[:, :D])
    z = _sigmoid(gi[:, D:2 * D] + gh[:, D:2 * D])
    n = jnp.tanh(gi[:, 2 * D:] + r * gh[:, 2 * D:])
    return (1.0 - z) * n + z * h


def _fuse_body(h_ref, mp_ref, mn_ref, hexp_ref,
               gpWx, gpbih, gpWh, gpbhh,
               gnWx, gnbih, gnWh, gnbhh,
               combWa, combWb, combb,
               gW1a, gW1b, gb1, gW2, gb2,
               fuseW, fuseb, fuseg, fusebb,
               out_ref):
    h = h_ref[0]
    hp = _gru(mp_ref[0], h, gpWx[...], gpbih[...], gpWh[...], gpbhh[...])
    hn = _gru(mn_ref[0], h, gnWx[...], gnbih[...], gnWh[...], gnbhh[...])
    h_imp = _mm(hp, combWa[...]) + _mm(hn, combWb[...]) + combb[...]
    h_exp = hexp_ref[0]
    gmid = jnp.maximum(_mm(h_imp, gW1a[...]) + _mm(h_exp, gW1b[...]) + gb1[...], 0.0)
    gate = _sigmoid(_mm(gmid, gW2[...]) + gb2[...])
    hf = gate * h_imp + (1.0 - gate) * h_exp
    out_ref[0] = jnp.maximum(_ln(_mm(hf, fuseW[...]) + fuseb[...], fuseg[...], fusebb[...]), 0.0)


def _row(a):
    return a.reshape(1, -1)


def _head_blockdiag(a):
    d = np.arange(H * O)[:, None] // O == np.arange(H)[None, :]
    return a.reshape(H * O, 1) * d.astype(np.float32)


def kernel(x_alpha, x_barra, money_flow_graph, sector_graph, params):
    p = params
    padr = lambda a: jnp.pad(a, ((0, 0), (0, NP - N), (0, 0)))
    xb = padr(x_barra[:, -1])
    xa = padr(x_alpha[:, -1])
    mfg = jnp.pad(money_flow_graph, ((0, 0), (0, NP - N), (0, NP - N)))
    secT = jnp.pad(jnp.swapaxes(sector_graph, 1, 2), ((0, 0), (0, NP - N), (0, NP - N)))

    bnd = lambda shp: jax.ShapeDtypeStruct(shp, jnp.float32)
    wspec = lambda a: pl.BlockSpec(a.shape, lambda b, *_: (0,) * a.ndim)
    bspec = lambda a: pl.BlockSpec((1,) + tuple(a.shape[1:]),
                                   lambda b, *_: (b,) + (0,) * (a.ndim - 1))
    bspec3 = pl.BlockSpec((1, NP, D), lambda b, *_: (b, 0, 0))

    pw = [p['risk_W'], _row(p['risk_b']), _row(p['risk_g']), _row(p['risk_bb']),
          p['attn_W1'][:D], p['attn_W1'][D:], _row(p['attn_b1']),
          p['msgp_W'], _row(p['msgp_b']), p['msgn_W'], _row(p['msgn_b']),
          p['in_W'], _row(p['in_b']), _row(p['in_g']), _row(p['in_bb'])]
    h, u, v, msgp, msgn, he = pl.pallas_call(
        _prep_body,
        grid=(B,),
        in_specs=[bspec(xb), bspec(xa)] + [wspec(w) for w in pw],
        out_specs=[bspec3] * 6,
        out_shape=[bnd((B, NP, D))] * 6,
    )(xb, xa, *pw)

    w2col = p['attn_W2']
    b2 = p['attn_b2'].reshape(1, 1)
    att = pl.pallas_call(
        _att_body,
        grid=(B, NT),
        in_specs=[pl.BlockSpec((1, R, D), lambda b, t: (b, t, 0)),
                  pl.BlockSpec((1, NP, D), lambda b, t: (b, 0, 0)),
                  pl.BlockSpec(w2col.shape, lambda b, t: (0, 0)),
                  pl.BlockSpec(b2.shape, lambda b, t: (0, 0))],
        out_specs=pl.BlockSpec((1, R, NP), lambda b, t: (b, t, 0)),
        out_shape=bnd((B, NP, NP)),
        compiler_params=pltpu.CompilerParams(
            dimension_semantics=("parallel", "arbitrary")),
    )(u, v, w2col, b2)

    padrow = lambda a: jnp.pad(a, ((0, NRPAD - NROWS), (0, 0)))
    mp2, mn2 = _sc_stage(padrow(att.reshape(NROWS, NP)),
                         padrow(mfg.reshape(NROWS, NP)),
                         msgp.reshape(NROWS, D), msgn.reshape(NROWS, D))
    mp = mp2[:NROWS].reshape(B, NP, D)
    mn = mn2[:NROWS].reshape(B, NP, D)

    gatw = [p['g0_W'], _head_blockdiag(p['g0_as']), _head_blockdiag(p['g0_ad']), _row(p['g0_b']),
            p['g1_W'], _head_blockdiag(p['g1_as']), _head_blockdiag(p['g1_ad']), _row(p['g1_b']),
            p['op_W'], _row(p['op_b'])]
    hexp = pl.pallas_call(
        _gat_body,
        grid=(B,),
        in_specs=[bspec(he), bspec(secT)] + [wspec(w) for w in gatw],
        out_specs=bspec3,
        out_shape=bnd((B, NP, D)),
    )(he, secT, *gatw)

    fw = [p['grup_Wx'], _row(p['grup_bih']), p['grup_Wh'], _row(p['grup_bhh']),
          p['grun_Wx'], _row(p['grun_bih']), p['grun_Wh'], _row(p['grun_bhh']),
          p['comb_W'][:D], p['comb_W'][D:], _row(p['comb_b']),
          p['gate_W1'][:D], p['gate_W1'][D:], _row(p['gate_b1']),
          p['gate_W2'], _row(p['gate_b2']),
          p['fuse_W'], _row(p['fuse_b']), _row(p['fuse_g']), _row(p['fuse_bb'])]
    out = pl.pallas_call(
        _fuse_body,
        grid=(B,),
        in_specs=[bspec(h), bspec(mp), bspec(mn), bspec(hexp)]
                 + [wspec(w) for w in fw],
        out_specs=bspec3,
        out_shape=bnd((B, NP, D)),
    )(h, mp, mn, hexp, *fw)
    return out[:, :N, :]
